# baseline (speedup 1.0000x reference)
import functools

import jax
import jax.numpy as jnp
from jax import lax
from jax.experimental import pallas as pl
from jax.experimental.pallas import tpu as pltpu
from jax.experimental.pallas import tpu_sc as plsc

NROWS = 1_000_000
DIM = 64
N_IDX = 16384 * 20
NUM_WORKERS = 32

TBLK = 8192
T_GRID = (NROWS + TBLK - 1) // TBLK


def _transpose_body(x_ref, o_ref):
    o_ref[...] = x_ref[...].T


def _transpose_tc(tt):
    return pl.pallas_call(
        _transpose_body,
        grid=(T_GRID,),
        in_specs=[pl.BlockSpec((DIM, TBLK), lambda b: (0, b))],
        out_specs=pl.BlockSpec((TBLK, DIM), lambda b: (b, 0)),
        out_shape=jax.ShapeDtypeStruct((NROWS, DIM), jnp.float32),
    )(tt)


B_PER_W = N_IDX // NUM_WORKERS
CHUNK = 128
N_CHUNKS = B_PER_W // CHUNK

_mesh = plsc.VectorSubcoreMesh(core_axis_name="c", subcore_axis_name="s")


@functools.partial(
    pl.kernel,
    mesh=_mesh,
    out_type=jax.ShapeDtypeStruct((N_IDX, DIM), jnp.float32),
    scratch_types=[
        pltpu.VMEM((B_PER_W,), jnp.int32),
        pltpu.VMEM((CHUNK, DIM), jnp.float32),
        pltpu.SemaphoreType.DMA,
    ],
    compiler_params=pltpu.CompilerParams(use_tc_tiling_on_sc=False),
)
def _gather_sc(idx_hbm, table_hbm, out_hbm, idx_v, rows_v, sem):
    wid = lax.axis_index("s") * 2 + lax.axis_index("c")
    base = wid * B_PER_W
    pltpu.sync_copy(idx_hbm.at[pl.ds(base, B_PER_W)], idx_v)

    def body(c, carry):
        off = c * CHUNK
        pltpu.async_copy(
            table_hbm.at[idx_v.at[pl.ds(off, CHUNK)]], rows_v, sem
        ).wait()
        pltpu.sync_copy(rows_v, out_hbm.at[pl.ds(base + off, CHUNK)])
        return carry

    lax.fori_loop(0, N_CHUNKS, body, 0)


def kernel(z, table):
    lin = _transpose_tc(table.T)
    out = _gather_sc(z.reshape(-1).astype(jnp.int32), lin)
    return (out.reshape(z.shape + (DIM,)), 0)


# speedup vs baseline: 1.0547x; 1.0547x over previous
import functools

import jax
import jax.numpy as jnp
from jax import lax
from jax.experimental import pallas as pl
from jax.experimental.pallas import tpu as pltpu
from jax.experimental.pallas import tpu_sc as plsc

NROWS = 1_000_000
DIM = 64
N_IDX = 16384 * 20
NUM_WORKERS = 32

TBLK = 8192
T_GRID = (NROWS + TBLK - 1) // TBLK


def _transpose_body(x_ref, o_ref):
    o_ref[...] = x_ref[...].T


def _transpose_tc(tt):
    return pl.pallas_call(
        _transpose_body,
        grid=(T_GRID,),
        in_specs=[pl.BlockSpec((DIM, TBLK), lambda b: (0, b))],
        out_specs=pl.BlockSpec((TBLK, DIM), lambda b: (b, 0)),
        out_shape=jax.ShapeDtypeStruct((NROWS, DIM), jnp.float32),
    )(tt)


B_PER_W = N_IDX // NUM_WORKERS
CHUNK = 640
N_CHUNKS = B_PER_W // CHUNK
NBUF = 2

_mesh = plsc.VectorSubcoreMesh(core_axis_name="c", subcore_axis_name="s")


@functools.partial(
    pl.kernel,
    mesh=_mesh,
    out_type=jax.ShapeDtypeStruct((N_IDX, DIM), jnp.float32),
    scratch_types=[
        pltpu.VMEM((B_PER_W,), jnp.int32),
        pltpu.VMEM((NBUF, CHUNK, DIM), jnp.float32),
        pltpu.SemaphoreType.DMA((NBUF,)),
        pltpu.SemaphoreType.DMA((NBUF,)),
    ],
    compiler_params=pltpu.CompilerParams(use_tc_tiling_on_sc=False),
)
def _gather_sc(idx_hbm, table_hbm, out_hbm, idx_v, rows_v, gsem, ssem):
    wid = lax.axis_index("s") * 2 + lax.axis_index("c")
    base = wid * B_PER_W
    pltpu.sync_copy(idx_hbm.at[pl.ds(base, B_PER_W)], idx_v)

    def gather(c, b):
        off = c * CHUNK
        return pltpu.async_copy(
            table_hbm.at[idx_v.at[pl.ds(off, CHUNK)]], rows_v.at[b], gsem.at[b]
        )

    def scatter(c, b):
        off = c * CHUNK
        return pltpu.async_copy(
            rows_v.at[b], out_hbm.at[pl.ds(base + off, CHUNK)], ssem.at[b]
        )

    gathers = [None] * N_CHUNKS
    scatters = [None] * N_CHUNKS
    gathers[0] = gather(0, 0)
    for c in range(N_CHUNKS):
        b = c % NBUF
        if c + 1 < N_CHUNKS:
            if c + 1 >= NBUF:
                scatters[c + 1 - NBUF].wait()
            gathers[c + 1] = gather(c + 1, (c + 1) % NBUF)
        gathers[c].wait()
        scatters[c] = scatter(c, b)
    for c in range(max(0, N_CHUNKS - NBUF), N_CHUNKS):
        scatters[c].wait()


def kernel(z, table):
    lin = _transpose_tc(table.T)
    out = _gather_sc(z.reshape(-1).astype(jnp.int32), lin)
    return (out.reshape(z.shape + (DIM,)), 0)


# speedup vs baseline: 1.0552x; 1.0004x over previous
import functools

import jax
import jax.numpy as jnp
from jax import lax
from jax.experimental import pallas as pl
from jax.experimental.pallas import tpu as pltpu
from jax.experimental.pallas import tpu_sc as plsc

NROWS = 1_000_000
DIM = 64
N_IDX = 16384 * 20
NUM_WORKERS = 32

TBLK = 8192
T_GRID = (NROWS + TBLK - 1) // TBLK


def _transpose_body(x_ref, o_ref):
    o_ref[...] = x_ref[...].T


def _transpose_tc(tt):
    return pl.pallas_call(
        _transpose_body,
        grid=(T_GRID,),
        in_specs=[pl.BlockSpec((DIM, TBLK), lambda b: (0, b))],
        out_specs=pl.BlockSpec((TBLK, DIM), lambda b: (b, 0)),
        out_shape=jax.ShapeDtypeStruct((NROWS, DIM), jnp.float32),
        compiler_params=pltpu.CompilerParams(
            dimension_semantics=("parallel",)
        ),
    )(tt)


B_PER_W = N_IDX // NUM_WORKERS
CHUNK = 640
N_CHUNKS = B_PER_W // CHUNK
NBUF = 2

_mesh = plsc.VectorSubcoreMesh(core_axis_name="c", subcore_axis_name="s")


@functools.partial(
    pl.kernel,
    mesh=_mesh,
    out_type=jax.ShapeDtypeStruct((N_IDX, DIM), jnp.float32),
    scratch_types=[
        pltpu.VMEM((B_PER_W,), jnp.int32),
        pltpu.VMEM((NBUF, CHUNK, DIM), jnp.float32),
        pltpu.SemaphoreType.DMA((NBUF,)),
        pltpu.SemaphoreType.DMA((NBUF,)),
    ],
    compiler_params=pltpu.CompilerParams(use_tc_tiling_on_sc=False),
)
def _gather_sc(idx_hbm, table_hbm, out_hbm, idx_v, rows_v, gsem, ssem):
    wid = lax.axis_index("s") * 2 + lax.axis_index("c")
    base = wid * B_PER_W
    pltpu.sync_copy(idx_hbm.at[pl.ds(base, B_PER_W)], idx_v)

    def gather(c, b):
        off = c * CHUNK
        return pltpu.async_copy(
            table_hbm.at[idx_v.at[pl.ds(off, CHUNK)]], rows_v.at[b], gsem.at[b]
        )

    def scatter(c, b):
        off = c * CHUNK
        return pltpu.async_copy(
            rows_v.at[b], out_hbm.at[pl.ds(base + off, CHUNK)], ssem.at[b]
        )

    gathers = [None] * N_CHUNKS
    scatters = [None] * N_CHUNKS
    gathers[0] = gather(0, 0)
    for c in range(N_CHUNKS):
        b = c % NBUF
        if c + 1 < N_CHUNKS:
            if c + 1 >= NBUF:
                scatters[c + 1 - NBUF].wait()
            gathers[c + 1] = gather(c + 1, (c + 1) % NBUF)
        gathers[c].wait()
        scatters[c] = scatter(c, b)
    for c in range(max(0, N_CHUNKS - NBUF), N_CHUNKS):
        scatters[c].wait()


def kernel(z, table):
    lin = _transpose_tc(table.T)
    out = _gather_sc(z.reshape(-1).astype(jnp.int32), lin)
    return (out.reshape(z.shape + (DIM,)), 0)


# speedup vs baseline: 1.0837x; 1.0270x over previous
import functools

import jax
import jax.numpy as jnp
from jax import lax
from jax.experimental import pallas as pl
from jax.experimental.pallas import tpu as pltpu
from jax.experimental.pallas import tpu_sc as plsc

NROWS = 1_000_000
DIM = 64
N_IDX = 16384 * 20
NUM_WORKERS = 32

TBLK = 32768
T_GRID = (NROWS + TBLK - 1) // TBLK


def _transpose_body(x_ref, o_ref):
    o_ref[...] = x_ref[...].T


def _transpose_tc(tt):
    return pl.pallas_call(
        _transpose_body,
        grid=(T_GRID,),
        in_specs=[pl.BlockSpec((DIM, TBLK), lambda b: (0, b))],
        out_specs=pl.BlockSpec((TBLK, DIM), lambda b: (b, 0)),
        out_shape=jax.ShapeDtypeStruct((NROWS, DIM), jnp.float32),
        compiler_params=pltpu.CompilerParams(
            dimension_semantics=("parallel",)
        ),
    )(tt)


B_PER_W = N_IDX // NUM_WORKERS
CHUNK = 640
N_CHUNKS = B_PER_W // CHUNK
NBUF = 2

_mesh = plsc.VectorSubcoreMesh(core_axis_name="c", subcore_axis_name="s")


@functools.partial(
    pl.kernel,
    mesh=_mesh,
    out_type=jax.ShapeDtypeStruct((N_IDX, DIM), jnp.float32),
    scratch_types=[
        pltpu.VMEM((B_PER_W,), jnp.int32),
        pltpu.VMEM((NBUF, CHUNK, DIM), jnp.float32),
        pltpu.SemaphoreType.DMA((NBUF,)),
        pltpu.SemaphoreType.DMA((NBUF,)),
    ],
    compiler_params=pltpu.CompilerParams(use_tc_tiling_on_sc=False),
)
def _gather_sc(idx_hbm, table_hbm, out_hbm, idx_v, rows_v, gsem, ssem):
    wid = lax.axis_index("s") * 2 + lax.axis_index("c")
    base = wid * B_PER_W
    pltpu.sync_copy(idx_hbm.at[pl.ds(base, B_PER_W)], idx_v)

    def gather(c, b):
        off = c * CHUNK
        return pltpu.async_copy(
            table_hbm.at[idx_v.at[pl.ds(off, CHUNK)]], rows_v.at[b], gsem.at[b]
        )

    def scatter(c, b):
        off = c * CHUNK
        return pltpu.async_copy(
            rows_v.at[b], out_hbm.at[pl.ds(base + off, CHUNK)], ssem.at[b]
        )

    gathers = [None] * N_CHUNKS
    scatters = [None] * N_CHUNKS
    gathers[0] = gather(0, 0)
    for c in range(N_CHUNKS):
        b = c % NBUF
        if c + 1 < N_CHUNKS:
            if c + 1 >= NBUF:
                scatters[c + 1 - NBUF].wait()
            gathers[c + 1] = gather(c + 1, (c + 1) % NBUF)
        gathers[c].wait()
        scatters[c] = scatter(c, b)
    for c in range(max(0, N_CHUNKS - NBUF), N_CHUNKS):
        scatters[c].wait()


def kernel(z, table):
    lin = _transpose_tc(table.T)
    out = _gather_sc(z.reshape(-1).astype(jnp.int32), lin)
    return (out.reshape(z.shape + (DIM,)), 0)
